# baseline (speedup 1.0000x reference)
import functools

import jax
import jax.numpy as jnp
from jax import lax
from jax.experimental import pallas as pl
from jax.experimental.pallas import tpu as pltpu
from jax.experimental.pallas import tpu_sc as plsc

_N = 100000
_E = 3200000
_NPAD = 102400
_NC = 2
_NS = 16
_K = 8
_CHUNK = _K * 128
_EPW = 200704
_EPAD = _EPW * _NS
_NROW = _NPAD // _NS
_F2 = 8


def _prop_body(ytab_hbm, src_hbm, dst_hbm, zeros_hbm, out_hbm,
               sidx, didx0, didx1, rows0, rows1, ysh, accsh, sem_g,
               sem_s0, sem_s1):
    c = lax.axis_index("c")
    s = lax.axis_index("s")
    rbase = s * _NROW
    pltpu.sync_copy(ytab_hbm.at[pl.ds(c * _NPAD + rbase, _NROW), :],
                    ysh.at[pl.ds(rbase, _NROW), :])
    pltpu.sync_copy(zeros_hbm.at[pl.ds(rbase, _NROW), :],
                    accsh.at[pl.ds(rbase, _NROW), :])
    plsc.subcore_barrier()

    idxrow0 = s * (_EPW // 128)
    bufs = ((rows0, didx0, sem_s0), (rows1, didx1, sem_s1))

    def _drain(rows, didx, sem):
        for j in range(_K):
            pltpu.make_async_copy(rows.at[pl.ds(j * 128, 128), :],
                                  accsh.at[didx.at[j]], sem).wait()

    def chunk2(g, carry):
        for b in range(2):
            rows, didx, sem = bufs[b]
            @pl.when(g > 0)
            def _():
                _drain(rows, didx, sem)

            r0 = idxrow0 + (g * 2 + b) * _K
            pltpu.sync_copy(src_hbm.at[pl.ds(r0, _K), :], sidx)
            pltpu.sync_copy(dst_hbm.at[pl.ds(r0, _K), :], didx)
            gets = [pltpu.async_copy(ysh.at[sidx.at[j]],
                                     rows.at[pl.ds(j * 128, 128), :], sem_g)
                    for j in range(_K)]
            for cp in gets:
                cp.wait()
            for j in range(_K):
                pltpu.async_copy(rows.at[pl.ds(j * 128, 128), :],
                                 accsh.at[didx.at[j]], sem, add=True)
        return carry

    lax.fori_loop(0, _EPW // _CHUNK // 2, chunk2, 0)
    for rows, didx, sem in bufs:
        _drain(rows, didx, sem)
    plsc.subcore_barrier()
    pltpu.sync_copy(accsh.at[pl.ds(rbase, _NROW), :],
                    out_hbm.at[pl.ds(c * _NPAD + rbase, _NROW), :])


_sc_propagate = functools.partial(
    pl.kernel,
    out_type=jax.ShapeDtypeStruct((_NC * _NPAD, _F2), jnp.float32),
    mesh=plsc.VectorSubcoreMesh(core_axis_name="c", subcore_axis_name="s",
                                num_cores=_NC, num_subcores=_NS),
    compiler_params=pltpu.CompilerParams(use_tc_tiling_on_sc=False),
    scratch_types=[
        pltpu.VMEM((_K, 128), jnp.int32),
        pltpu.VMEM((_K, 128), jnp.int32),
        pltpu.VMEM((_K, 128), jnp.int32),
        pltpu.VMEM((_CHUNK, _F2), jnp.float32),
        pltpu.VMEM((_CHUNK, _F2), jnp.float32),
        pltpu.VMEM_SHARED((_NPAD, _F2), jnp.float32),
        pltpu.VMEM_SHARED((_NPAD, _F2), jnp.float32),
        pltpu.SemaphoreType.DMA,
        pltpu.SemaphoreType.DMA,
        pltpu.SemaphoreType.DMA,
    ],
)(_prop_body)


def _deg_body(dst_hbm, zeros_hbm, out_hbm, didx, ones_v, degsh, sem_s):
    c = lax.axis_index("c")
    s = lax.axis_index("s")
    rbase = s * _NROW
    pltpu.sync_copy(zeros_hbm.at[pl.ds(rbase, _NROW)],
                    degsh.at[pl.ds(rbase, _NROW)])
    for j in range(8):
        ones_v[pl.ds(j * 16, 16)] = jnp.ones((16,), jnp.float32)
    plsc.subcore_barrier()

    epw = _EPAD // (_NC * _NS)
    idxrow0 = (c * _NS + s) * (epw // 128)

    def chunk(i, carry):
        r0 = idxrow0 + i * _K
        pltpu.sync_copy(dst_hbm.at[pl.ds(r0, _K), :], didx)
        puts = [pltpu.async_copy(ones_v, degsh.at[didx.at[j]], sem_s,
                                 add=True)
                for j in range(_K)]
        for cp in puts:
            cp.wait()
        return carry

    lax.fori_loop(0, epw // _CHUNK, chunk, 0)
    plsc.subcore_barrier()
    pltpu.sync_copy(degsh.at[pl.ds(rbase, _NROW)],
                    out_hbm.at[pl.ds(c * _NPAD + rbase, _NROW)])


_sc_degree = functools.partial(
    pl.kernel,
    out_type=jax.ShapeDtypeStruct((_NC * _NPAD,), jnp.float32),
    mesh=plsc.VectorSubcoreMesh(core_axis_name="c", subcore_axis_name="s",
                                num_cores=_NC, num_subcores=_NS),
    compiler_params=pltpu.CompilerParams(use_tc_tiling_on_sc=False),
    scratch_types=[
        pltpu.VMEM((_K, 128), jnp.int32),
        pltpu.VMEM((128,), jnp.float32),
        pltpu.VMEM_SHARED((_NPAD,), jnp.float32),
        pltpu.SemaphoreType.DMA,
    ],
)(_deg_body)


_RB = 2048
_GRID = _NPAD // _RB


def _tc1_body(indeg_ref, x_ref, dinv_ref, y1_ref):
    p = indeg_ref[...]
    deg = (p[0] + p[1] + 1.0)[:, None]
    dv = lax.rsqrt(deg)
    dinv_ref[...] = dv
    xv = x_ref[...]
    y1_ref[0, :, :] = xv[:, :8] * dv
    y1_ref[1, :, :] = xv[:, 8:] * dv


def _tc2_body(acc_ref, y_ref, dinv_ref, w_ref, b_ref, y2a_ref, y2b_ref):
    a = acc_ref[...]
    y = y_ref[...]
    dv = dinv_ref[...]
    m = jnp.concatenate([a[0] + y[0], a[1] + y[1]], axis=1) * dv
    h = jnp.maximum(
        jnp.dot(m, w_ref[...], preferred_element_type=jnp.float32)
        + b_ref[...], 0.0)
    rows = (jax.lax.broadcasted_iota(jnp.int32, (_RB, 1), 0)
            + pl.program_id(0) * _RB)
    yh = h * dv * (rows < _N).astype(jnp.float32)
    y2a_ref[0, :, :] = yh[:, 0:8]
    y2a_ref[1, :, :] = yh[:, 8:16]
    y2b_ref[0, :, :] = yh[:, 16:24]
    y2b_ref[1, :, :] = yh[:, 24:32]


def _tc3_body(acca_ref, accb_ref, ya_ref, yb_ref, dinv_ref, w_ref, b_ref,
              wfc_ref, bfc_ref, gsum_ref, out_ref):
    i = pl.program_id(0)

    @pl.when(i == 0)
    def _():
        gsum_ref[...] = jnp.zeros_like(gsum_ref)

    a = acca_ref[...]
    b = accb_ref[...]
    ya = ya_ref[...]
    yb = yb_ref[...]
    dv = dinv_ref[...]
    m = jnp.concatenate([a[0] + ya[0], a[1] + ya[1],
                         b[0] + yb[0], b[1] + yb[1]], axis=1) * dv
    h = jnp.maximum(
        jnp.dot(m, w_ref[...], preferred_element_type=jnp.float32)
        + b_ref[...], 0.0)
    rows = (jax.lax.broadcasted_iota(jnp.int32, (_RB, 1), 0) + i * _RB)
    mask = (rows < _N).astype(jnp.float32)
    gsum_ref[...] += jnp.sum(h * mask, axis=0, keepdims=True)

    @pl.when(i == _GRID - 1)
    def _():
        g = gsum_ref[...] * (1.0 / _N)
        out_ref[...] = (jnp.dot(g, wfc_ref[...],
                                preferred_element_type=jnp.float32)
                        + bfc_ref[...])


def kernel(x, edge_index, W1, b1, W2, b2, Wfc, bfc):
    f32 = jnp.float32
    src = edge_index[0]
    dst = edge_index[1]
    pad_e = _EPAD - _E
    fill = jnp.full((pad_e,), _N, jnp.int32)
    src2d = jnp.concatenate([src, fill]).reshape(_EPAD // 128, 128)
    dst2d = jnp.concatenate([dst, fill]).reshape(_EPAD // 128, 128)
    x_pad = jnp.pad(x, ((0, _NPAD - _N), (0, 0)))
    zeros8 = jnp.zeros((_NPAD, _F2), f32)
    zeros1 = jnp.zeros((_NPAD,), f32)

    indeg = _sc_degree(dst2d, zeros1).reshape(2, _NPAD)

    dinv, y1 = pl.pallas_call(
        _tc1_body,
        grid=(_GRID,),
        in_specs=[
            pl.BlockSpec((2, _RB), lambda i: (0, i)),
            pl.BlockSpec((_RB, 16), lambda i: (i, 0)),
        ],
        out_specs=[
            pl.BlockSpec((_RB, 1), lambda i: (i, 0)),
            pl.BlockSpec((2, _RB, 8), lambda i: (0, i, 0)),
        ],
        out_shape=[
            jax.ShapeDtypeStruct((_NPAD, 1), f32),
            jax.ShapeDtypeStruct((2, _NPAD, 8), f32),
        ],
    )(indeg, x_pad)

    acc1 = _sc_propagate(y1.reshape(_NC * _NPAD, _F2), src2d, dst2d, zeros8)

    y2a, y2b = pl.pallas_call(
        _tc2_body,
        grid=(_GRID,),
        in_specs=[
            pl.BlockSpec((2, _RB, 8), lambda i: (0, i, 0)),
            pl.BlockSpec((2, _RB, 8), lambda i: (0, i, 0)),
            pl.BlockSpec((_RB, 1), lambda i: (i, 0)),
            pl.BlockSpec((16, 32), lambda i: (0, 0)),
            pl.BlockSpec((1, 32), lambda i: (0, 0)),
        ],
        out_specs=[
            pl.BlockSpec((2, _RB, 8), lambda i: (0, i, 0)),
            pl.BlockSpec((2, _RB, 8), lambda i: (0, i, 0)),
        ],
        out_shape=[
            jax.ShapeDtypeStruct((2, _NPAD, 8), f32),
            jax.ShapeDtypeStruct((2, _NPAD, 8), f32),
        ],
    )(acc1.reshape(2, _NPAD, 8), y1, dinv, W1, b1.reshape(1, 32))

    acc2a = _sc_propagate(y2a.reshape(_NC * _NPAD, _F2), src2d, dst2d,
                          zeros8)
    acc2b = _sc_propagate(y2b.reshape(_NC * _NPAD, _F2), src2d, dst2d,
                          zeros8)

    _, out = pl.pallas_call(
        _tc3_body,
        grid=(_GRID,),
        in_specs=[
            pl.BlockSpec((2, _RB, 8), lambda i: (0, i, 0)),
            pl.BlockSpec((2, _RB, 8), lambda i: (0, i, 0)),
            pl.BlockSpec((2, _RB, 8), lambda i: (0, i, 0)),
            pl.BlockSpec((2, _RB, 8), lambda i: (0, i, 0)),
            pl.BlockSpec((_RB, 1), lambda i: (i, 0)),
            pl.BlockSpec((32, 64), lambda i: (0, 0)),
            pl.BlockSpec((1, 64), lambda i: (0, 0)),
            pl.BlockSpec((64, 1), lambda i: (0, 0)),
            pl.BlockSpec((1, 1), lambda i: (0, 0)),
        ],
        out_specs=[
            pl.BlockSpec((1, 64), lambda i: (0, 0)),
            pl.BlockSpec((1, 1), lambda i: (0, 0)),
        ],
        out_shape=[
            jax.ShapeDtypeStruct((1, 64), f32),
            jax.ShapeDtypeStruct((1, 1), f32),
        ],
    )(acc2a.reshape(2, _NPAD, 8), acc2b.reshape(2, _NPAD, 8), y2a, y2b,
      dinv, W2, b2.reshape(1, 64), Wfc, bfc.reshape(1, 1))

    return out.reshape(1)


# speedup vs baseline: 1.0970x; 1.0970x over previous
import functools

import jax
import jax.numpy as jnp
from jax import lax
from jax.experimental import pallas as pl
from jax.experimental.pallas import tpu as pltpu
from jax.experimental.pallas import tpu_sc as plsc

_N = 100000
_E = 3200000
_NPAD = 102400
_NC = 2
_NS = 16
_K = 16
_CHUNK = _K * 128
_EPW = 200704
_EPAD = _EPW * _NS
_NROW = _NPAD // _NS
_F2 = 8


def _prop_body(ytab_hbm, src_hbm, dst_hbm, zeros_hbm, out_hbm,
               sidx, didx, rows, ysh, accsh, sem_g, sem_s):
    c = lax.axis_index("c")
    s = lax.axis_index("s")
    rbase = s * _NROW
    pltpu.sync_copy(ytab_hbm.at[pl.ds(c * _NPAD + rbase, _NROW), :],
                    ysh.at[pl.ds(rbase, _NROW), :])
    pltpu.sync_copy(zeros_hbm.at[pl.ds(rbase, _NROW), :],
                    accsh.at[pl.ds(rbase, _NROW), :])
    plsc.subcore_barrier()

    idxrow0 = s * (_EPW // 128)

    def chunk(i, carry):
        r0 = idxrow0 + i * _K
        pltpu.sync_copy(src_hbm.at[pl.ds(r0, _K), :], sidx)
        pltpu.sync_copy(dst_hbm.at[pl.ds(r0, _K), :], didx)
        gets = [pltpu.async_copy(ysh.at[sidx.at[j]],
                                 rows.at[pl.ds(j * 128, 128), :], sem_g)
                for j in range(_K)]
        for j in range(_K):
            gets[j].wait()
            pltpu.async_copy(rows.at[pl.ds(j * 128, 128), :],
                             accsh.at[didx.at[j]], sem_s, add=True)
        for j in range(_K):
            pltpu.make_async_copy(rows.at[pl.ds(j * 128, 128), :],
                                  accsh.at[didx.at[j]], sem_s).wait()
        return carry

    lax.fori_loop(0, _EPW // _CHUNK, chunk, 0)
    plsc.subcore_barrier()
    pltpu.sync_copy(accsh.at[pl.ds(rbase, _NROW), :],
                    out_hbm.at[pl.ds(c * _NPAD + rbase, _NROW), :])


_sc_propagate = functools.partial(
    pl.kernel,
    out_type=jax.ShapeDtypeStruct((_NC * _NPAD, _F2), jnp.float32),
    mesh=plsc.VectorSubcoreMesh(core_axis_name="c", subcore_axis_name="s",
                                num_cores=_NC, num_subcores=_NS),
    compiler_params=pltpu.CompilerParams(use_tc_tiling_on_sc=False),
    scratch_types=[
        pltpu.VMEM((_K, 128), jnp.int32),
        pltpu.VMEM((_K, 128), jnp.int32),
        pltpu.VMEM((_CHUNK, _F2), jnp.float32),
        pltpu.VMEM_SHARED((_NPAD, _F2), jnp.float32),
        pltpu.VMEM_SHARED((_NPAD, _F2), jnp.float32),
        pltpu.SemaphoreType.DMA,
        pltpu.SemaphoreType.DMA,
    ],
)(_prop_body)


def _deg_body(dst_hbm, zeros_hbm, out_hbm, didx, ones_v, degsh, sem_s):
    c = lax.axis_index("c")
    s = lax.axis_index("s")
    rbase = s * _NROW
    pltpu.sync_copy(zeros_hbm.at[pl.ds(rbase, _NROW)],
                    degsh.at[pl.ds(rbase, _NROW)])
    for j in range(8):
        ones_v[pl.ds(j * 16, 16)] = jnp.ones((16,), jnp.float32)
    plsc.subcore_barrier()

    epw = _EPAD // (_NC * _NS)
    idxrow0 = (c * _NS + s) * (epw // 128)

    def chunk(i, carry):
        r0 = idxrow0 + i * _K
        pltpu.sync_copy(dst_hbm.at[pl.ds(r0, _K), :], didx)
        puts = [pltpu.async_copy(ones_v, degsh.at[didx.at[j]], sem_s,
                                 add=True)
                for j in range(_K)]
        for cp in puts:
            cp.wait()
        return carry

    lax.fori_loop(0, epw // _CHUNK, chunk, 0)
    plsc.subcore_barrier()
    pltpu.sync_copy(degsh.at[pl.ds(rbase, _NROW)],
                    out_hbm.at[pl.ds(c * _NPAD + rbase, _NROW)])


_sc_degree = functools.partial(
    pl.kernel,
    out_type=jax.ShapeDtypeStruct((_NC * _NPAD,), jnp.float32),
    mesh=plsc.VectorSubcoreMesh(core_axis_name="c", subcore_axis_name="s",
                                num_cores=_NC, num_subcores=_NS),
    compiler_params=pltpu.CompilerParams(use_tc_tiling_on_sc=False),
    scratch_types=[
        pltpu.VMEM((_K, 128), jnp.int32),
        pltpu.VMEM((128,), jnp.float32),
        pltpu.VMEM_SHARED((_NPAD,), jnp.float32),
        pltpu.SemaphoreType.DMA,
    ],
)(_deg_body)


_RB = 2048
_GRID = _NPAD // _RB


def _tc1_body(indeg_ref, x_ref, dinv_ref, y1_ref):
    p = indeg_ref[...]
    deg = (p[0] + p[1] + 1.0)[:, None]
    dv = lax.rsqrt(deg)
    dinv_ref[...] = dv
    xv = x_ref[...]
    y1_ref[0, :, :] = xv[:, :8] * dv
    y1_ref[1, :, :] = xv[:, 8:] * dv


def _tc2_body(acc_ref, y_ref, dinv_ref, w_ref, b_ref, y2a_ref, y2b_ref):
    a = acc_ref[...]
    y = y_ref[...]
    dv = dinv_ref[...]
    m = jnp.concatenate([a[0] + y[0], a[1] + y[1]], axis=1) * dv
    h = jnp.maximum(
        jnp.dot(m, w_ref[...], preferred_element_type=jnp.float32)
        + b_ref[...], 0.0)
    rows = (jax.lax.broadcasted_iota(jnp.int32, (_RB, 1), 0)
            + pl.program_id(0) * _RB)
    yh = h * dv * (rows < _N).astype(jnp.float32)
    y2a_ref[0, :, :] = yh[:, 0:8]
    y2a_ref[1, :, :] = yh[:, 8:16]
    y2b_ref[0, :, :] = yh[:, 16:24]
    y2b_ref[1, :, :] = yh[:, 24:32]


def _tc3_body(acca_ref, accb_ref, ya_ref, yb_ref, dinv_ref, w_ref, b_ref,
              wfc_ref, bfc_ref, gsum_ref, out_ref):
    i = pl.program_id(0)

    @pl.when(i == 0)
    def _():
        gsum_ref[...] = jnp.zeros_like(gsum_ref)

    a = acca_ref[...]
    b = accb_ref[...]
    ya = ya_ref[...]
    yb = yb_ref[...]
    dv = dinv_ref[...]
    m = jnp.concatenate([a[0] + ya[0], a[1] + ya[1],
                         b[0] + yb[0], b[1] + yb[1]], axis=1) * dv
    h = jnp.maximum(
        jnp.dot(m, w_ref[...], preferred_element_type=jnp.float32)
        + b_ref[...], 0.0)
    rows = (jax.lax.broadcasted_iota(jnp.int32, (_RB, 1), 0) + i * _RB)
    mask = (rows < _N).astype(jnp.float32)
    gsum_ref[...] += jnp.sum(h * mask, axis=0, keepdims=True)

    @pl.when(i == _GRID - 1)
    def _():
        g = gsum_ref[...] * (1.0 / _N)
        out_ref[...] = (jnp.dot(g, wfc_ref[...],
                                preferred_element_type=jnp.float32)
                        + bfc_ref[...])


def kernel(x, edge_index, W1, b1, W2, b2, Wfc, bfc):
    f32 = jnp.float32
    src = edge_index[0]
    dst = edge_index[1]
    pad_e = _EPAD - _E
    fill = jnp.full((pad_e,), _N, jnp.int32)
    src2d = jnp.concatenate([src, fill]).reshape(_EPAD // 128, 128)
    dst2d = jnp.concatenate([dst, fill]).reshape(_EPAD // 128, 128)
    x_pad = jnp.pad(x, ((0, _NPAD - _N), (0, 0)))
    zeros8 = jnp.zeros((_NPAD, _F2), f32)
    zeros1 = jnp.zeros((_NPAD,), f32)

    indeg = _sc_degree(dst2d, zeros1).reshape(2, _NPAD)

    dinv, y1 = pl.pallas_call(
        _tc1_body,
        grid=(_GRID,),
        in_specs=[
            pl.BlockSpec((2, _RB), lambda i: (0, i)),
            pl.BlockSpec((_RB, 16), lambda i: (i, 0)),
        ],
        out_specs=[
            pl.BlockSpec((_RB, 1), lambda i: (i, 0)),
            pl.BlockSpec((2, _RB, 8), lambda i: (0, i, 0)),
        ],
        out_shape=[
            jax.ShapeDtypeStruct((_NPAD, 1), f32),
            jax.ShapeDtypeStruct((2, _NPAD, 8), f32),
        ],
    )(indeg, x_pad)

    acc1 = _sc_propagate(y1.reshape(_NC * _NPAD, _F2), src2d, dst2d, zeros8)

    y2a, y2b = pl.pallas_call(
        _tc2_body,
        grid=(_GRID,),
        in_specs=[
            pl.BlockSpec((2, _RB, 8), lambda i: (0, i, 0)),
            pl.BlockSpec((2, _RB, 8), lambda i: (0, i, 0)),
            pl.BlockSpec((_RB, 1), lambda i: (i, 0)),
            pl.BlockSpec((16, 32), lambda i: (0, 0)),
            pl.BlockSpec((1, 32), lambda i: (0, 0)),
        ],
        out_specs=[
            pl.BlockSpec((2, _RB, 8), lambda i: (0, i, 0)),
            pl.BlockSpec((2, _RB, 8), lambda i: (0, i, 0)),
        ],
        out_shape=[
            jax.ShapeDtypeStruct((2, _NPAD, 8), f32),
            jax.ShapeDtypeStruct((2, _NPAD, 8), f32),
        ],
    )(acc1.reshape(2, _NPAD, 8), y1, dinv, W1, b1.reshape(1, 32))

    acc2a = _sc_propagate(y2a.reshape(_NC * _NPAD, _F2), src2d, dst2d,
                          zeros8)
    acc2b = _sc_propagate(y2b.reshape(_NC * _NPAD, _F2), src2d, dst2d,
                          zeros8)

    _, out = pl.pallas_call(
        _tc3_body,
        grid=(_GRID,),
        in_specs=[
            pl.BlockSpec((2, _RB, 8), lambda i: (0, i, 0)),
            pl.BlockSpec((2, _RB, 8), lambda i: (0, i, 0)),
            pl.BlockSpec((2, _RB, 8), lambda i: (0, i, 0)),
            pl.BlockSpec((2, _RB, 8), lambda i: (0, i, 0)),
            pl.BlockSpec((_RB, 1), lambda i: (i, 0)),
            pl.BlockSpec((32, 64), lambda i: (0, 0)),
            pl.BlockSpec((1, 64), lambda i: (0, 0)),
            pl.BlockSpec((64, 1), lambda i: (0, 0)),
            pl.BlockSpec((1, 1), lambda i: (0, 0)),
        ],
        out_specs=[
            pl.BlockSpec((1, 64), lambda i: (0, 0)),
            pl.BlockSpec((1, 1), lambda i: (0, 0)),
        ],
        out_shape=[
            jax.ShapeDtypeStruct((1, 64), f32),
            jax.ShapeDtypeStruct((1, 1), f32),
        ],
    )(acc2a.reshape(2, _NPAD, 8), acc2b.reshape(2, _NPAD, 8), y2a, y2b,
      dinv, W2, b2.reshape(1, 64), Wfc, bfc.reshape(1, 1))

    return out.reshape(1)


# speedup vs baseline: 1.1721x; 1.0685x over previous
import functools

import jax
import jax.numpy as jnp
from jax import lax
from jax.experimental import pallas as pl
from jax.experimental.pallas import tpu as pltpu
from jax.experimental.pallas import tpu_sc as plsc

_N = 100000
_E = 3200000
_NPAD = 102400
_NC = 2
_NS = 16
_K = 16
_CHUNK = _K * 128
_EPW = 200704
_EPAD = _EPW * _NS
_NROW = _NPAD // _NS
_F2 = 8


def _prop_body(ytab_hbm, sd_hbm, zeros_hbm, out_hbm,
               sdidx, rows, ysh, accsh, sem_g, sem_s):
    c = lax.axis_index("c")
    s = lax.axis_index("s")
    rbase = s * _NROW
    pltpu.sync_copy(ytab_hbm.at[pl.ds(c * _NPAD + rbase, _NROW), :],
                    ysh.at[pl.ds(rbase, _NROW), :])
    pltpu.sync_copy(zeros_hbm.at[pl.ds(rbase, _NROW), :],
                    accsh.at[pl.ds(rbase, _NROW), :])
    plsc.subcore_barrier()

    idxrow0 = s * (_EPW // 128)

    def chunk(i, carry):
        r0 = idxrow0 + i * _K
        pltpu.sync_copy(sd_hbm.at[pl.ds(r0, _K), :, :], sdidx)
        gets = [pltpu.async_copy(ysh.at[sdidx.at[j, 0]],
                                 rows.at[pl.ds(j * 128, 128), :], sem_g)
                for j in range(_K)]
        for j in range(_K):
            gets[j].wait()
            pltpu.async_copy(rows.at[pl.ds(j * 128, 128), :],
                             accsh.at[sdidx.at[j, 1]], sem_s, add=True)
        for j in range(_K):
            pltpu.make_async_copy(rows.at[pl.ds(j * 128, 128), :],
                                  accsh.at[sdidx.at[j, 1]], sem_s).wait()
        return carry

    lax.fori_loop(0, _EPW // _CHUNK, chunk, 0)
    plsc.subcore_barrier()
    pltpu.sync_copy(accsh.at[pl.ds(rbase, _NROW), :],
                    out_hbm.at[pl.ds(c * _NPAD + rbase, _NROW), :])


_sc_propagate = functools.partial(
    pl.kernel,
    out_type=jax.ShapeDtypeStruct((_NC * _NPAD, _F2), jnp.float32),
    mesh=plsc.VectorSubcoreMesh(core_axis_name="c", subcore_axis_name="s",
                                num_cores=_NC, num_subcores=_NS),
    compiler_params=pltpu.CompilerParams(use_tc_tiling_on_sc=False),
    scratch_types=[
        pltpu.VMEM((_K, 2, 128), jnp.int32),
        pltpu.VMEM((_CHUNK, _F2), jnp.float32),
        pltpu.VMEM_SHARED((_NPAD, _F2), jnp.float32),
        pltpu.VMEM_SHARED((_NPAD, _F2), jnp.float32),
        pltpu.SemaphoreType.DMA,
        pltpu.SemaphoreType.DMA,
    ],
)(_prop_body)


def _deg_body(dst_hbm, zeros_hbm, out_hbm, didx, ones_v, degsh, sem_s):
    c = lax.axis_index("c")
    s = lax.axis_index("s")
    rbase = s * _NROW
    pltpu.sync_copy(zeros_hbm.at[pl.ds(rbase, _NROW)],
                    degsh.at[pl.ds(rbase, _NROW)])
    for j in range(8):
        ones_v[pl.ds(j * 16, 16)] = jnp.ones((16,), jnp.float32)
    plsc.subcore_barrier()

    epw = _EPAD // (_NC * _NS)
    idxrow0 = (c * _NS + s) * (epw // 128)

    def chunk(i, carry):
        r0 = idxrow0 + i * _K
        pltpu.sync_copy(dst_hbm.at[pl.ds(r0, _K), :], didx)
        for j in range(_K):
            pltpu.async_copy(ones_v, degsh.at[didx.at[j]], sem_s, add=True)
        for j in range(_K):
            pltpu.make_async_copy(ones_v, degsh.at[didx.at[j]], sem_s).wait()
        return carry

    lax.fori_loop(0, epw // _CHUNK, chunk, 0)
    plsc.subcore_barrier()
    pltpu.sync_copy(degsh.at[pl.ds(rbase, _NROW)],
                    out_hbm.at[pl.ds(c * _NPAD + rbase, _NROW)])


_sc_degree = functools.partial(
    pl.kernel,
    out_type=jax.ShapeDtypeStruct((_NC * _NPAD,), jnp.float32),
    mesh=plsc.VectorSubcoreMesh(core_axis_name="c", subcore_axis_name="s",
                                num_cores=_NC, num_subcores=_NS),
    compiler_params=pltpu.CompilerParams(use_tc_tiling_on_sc=False),
    scratch_types=[
        pltpu.VMEM((_K, 128), jnp.int32),
        pltpu.VMEM((128,), jnp.float32),
        pltpu.VMEM_SHARED((_NPAD,), jnp.float32),
        pltpu.SemaphoreType.DMA,
    ],
)(_deg_body)


_RB = 2048
_GRID = _NPAD // _RB


def _tc1_body(indeg_ref, x_ref, dinv_ref, y1_ref):
    p = indeg_ref[...]
    deg = (p[0] + p[1] + 1.0)[:, None]
    dv = 1.0 / jnp.sqrt(deg)
    dinv_ref[...] = dv
    xv = x_ref[...]
    y1_ref[0, :, :] = xv[:, :8] * dv
    y1_ref[1, :, :] = xv[:, 8:] * dv


def _tc2_body(acc_ref, y_ref, dinv_ref, w_ref, b_ref, y2a_ref, y2b_ref):
    a = acc_ref[...]
    y = y_ref[...]
    dv = dinv_ref[...]
    m = jnp.concatenate([a[0] + y[0], a[1] + y[1]], axis=1) * dv
    h = jnp.maximum(
        jnp.dot(m, w_ref[...], preferred_element_type=jnp.float32)
        + b_ref[...], 0.0)
    rows = (jax.lax.broadcasted_iota(jnp.int32, (_RB, 1), 0)
            + pl.program_id(0) * _RB)
    yh = h * dv * (rows < _N).astype(jnp.float32)
    y2a_ref[0, :, :] = yh[:, 0:8]
    y2a_ref[1, :, :] = yh[:, 8:16]
    y2b_ref[0, :, :] = yh[:, 16:24]
    y2b_ref[1, :, :] = yh[:, 24:32]


def _tc3_body(acca_ref, accb_ref, ya_ref, yb_ref, dinv_ref, w_ref, b_ref,
              wfc_ref, bfc_ref, gsum_ref, out_ref):
    i = pl.program_id(0)

    @pl.when(i == 0)
    def _():
        gsum_ref[...] = jnp.zeros_like(gsum_ref)

    a = acca_ref[...]
    b = accb_ref[...]
    ya = ya_ref[...]
    yb = yb_ref[...]
    dv = dinv_ref[...]
    m = jnp.concatenate([a[0] + ya[0], a[1] + ya[1],
                         b[0] + yb[0], b[1] + yb[1]], axis=1) * dv
    h = jnp.maximum(
        jnp.dot(m, w_ref[...], preferred_element_type=jnp.float32)
        + b_ref[...], 0.0)
    rows = (jax.lax.broadcasted_iota(jnp.int32, (_RB, 1), 0) + i * _RB)
    mask = (rows < _N).astype(jnp.float32)
    gsum_ref[...] += jnp.sum(h * mask, axis=0, keepdims=True)

    @pl.when(i == _GRID - 1)
    def _():
        g = gsum_ref[...] * (1.0 / _N)
        out_ref[...] = (jnp.dot(g, wfc_ref[...],
                                preferred_element_type=jnp.float32)
                        + bfc_ref[...])


def kernel(x, edge_index, W1, b1, W2, b2, Wfc, bfc):
    f32 = jnp.float32
    src = edge_index[0]
    dst = edge_index[1]
    pad_e = _EPAD - _E
    fill = jnp.full((pad_e,), _N, jnp.int32)
    src2d = jnp.concatenate([src, fill]).reshape(_EPAD // 128, 128)
    dst2d = jnp.concatenate([dst, fill]).reshape(_EPAD // 128, 128)
    sd2d = jnp.stack([src2d, dst2d], axis=1)
    x_pad = jnp.pad(x, ((0, _NPAD - _N), (0, 0)))
    zeros8 = jnp.zeros((_NPAD, _F2), f32)
    zeros1 = jnp.zeros((_NPAD,), f32)

    indeg = _sc_degree(dst2d, zeros1).reshape(2, _NPAD)

    dinv, y1 = pl.pallas_call(
        _tc1_body,
        grid=(_GRID,),
        in_specs=[
            pl.BlockSpec((2, _RB), lambda i: (0, i)),
            pl.BlockSpec((_RB, 16), lambda i: (i, 0)),
        ],
        out_specs=[
            pl.BlockSpec((_RB, 1), lambda i: (i, 0)),
            pl.BlockSpec((2, _RB, 8), lambda i: (0, i, 0)),
        ],
        out_shape=[
            jax.ShapeDtypeStruct((_NPAD, 1), f32),
            jax.ShapeDtypeStruct((2, _NPAD, 8), f32),
        ],
    )(indeg, x_pad)

    acc1 = _sc_propagate(y1.reshape(_NC * _NPAD, _F2), sd2d, zeros8)

    y2a, y2b = pl.pallas_call(
        _tc2_body,
        grid=(_GRID,),
        in_specs=[
            pl.BlockSpec((2, _RB, 8), lambda i: (0, i, 0)),
            pl.BlockSpec((2, _RB, 8), lambda i: (0, i, 0)),
            pl.BlockSpec((_RB, 1), lambda i: (i, 0)),
            pl.BlockSpec((16, 32), lambda i: (0, 0)),
            pl.BlockSpec((1, 32), lambda i: (0, 0)),
        ],
        out_specs=[
            pl.BlockSpec((2, _RB, 8), lambda i: (0, i, 0)),
            pl.BlockSpec((2, _RB, 8), lambda i: (0, i, 0)),
        ],
        out_shape=[
            jax.ShapeDtypeStruct((2, _NPAD, 8), f32),
            jax.ShapeDtypeStruct((2, _NPAD, 8), f32),
        ],
    )(acc1.reshape(2, _NPAD, 8), y1, dinv, W1, b1.reshape(1, 32))

    acc2a = _sc_propagate(y2a.reshape(_NC * _NPAD, _F2), sd2d, zeros8)
    acc2b = _sc_propagate(y2b.reshape(_NC * _NPAD, _F2), sd2d, zeros8)

    _, out = pl.pallas_call(
        _tc3_body,
        grid=(_GRID,),
        in_specs=[
            pl.BlockSpec((2, _RB, 8), lambda i: (0, i, 0)),
            pl.BlockSpec((2, _RB, 8), lambda i: (0, i, 0)),
            pl.BlockSpec((2, _RB, 8), lambda i: (0, i, 0)),
            pl.BlockSpec((2, _RB, 8), lambda i: (0, i, 0)),
            pl.BlockSpec((_RB, 1), lambda i: (i, 0)),
            pl.BlockSpec((32, 64), lambda i: (0, 0)),
            pl.BlockSpec((1, 64), lambda i: (0, 0)),
            pl.BlockSpec((64, 1), lambda i: (0, 0)),
            pl.BlockSpec((1, 1), lambda i: (0, 0)),
        ],
        out_specs=[
            pl.BlockSpec((1, 64), lambda i: (0, 0)),
            pl.BlockSpec((1, 1), lambda i: (0, 0)),
        ],
        out_shape=[
            jax.ShapeDtypeStruct((1, 64), f32),
            jax.ShapeDtypeStruct((1, 1), f32),
        ],
    )(acc2a.reshape(2, _NPAD, 8), acc2b.reshape(2, _NPAD, 8), y2a, y2b,
      dinv, W2, b2.reshape(1, 64), Wfc, bfc.reshape(1, 1))

    return out.reshape(1)
